# Initial kernel scaffold; baseline (speedup 1.0000x reference)
import functools

import jax
import jax.numpy as jnp
from jax import lax
from jax.experimental import pallas as pl
from jax.experimental.pallas import tpu as pltpu
from jax.experimental.pallas import tpu_sc as plsc

N = 10000
G = 64
H = 512
EPS = 1e-5
C = 128
LB = 128
NT = 16
NC = 2
NB_E = 80
EPT = NB_E * LB
EPAD = EPT * NT
ACC_ROWS = 10016
ZR = ACC_ROWS // NT
WR = N // NT
BR = 1000
NBLK = N // BR


@functools.lru_cache(None)
def _segsum_call(K):
    Khalf = K // 2
    mesh = plsc.VectorSubcoreMesh(core_axis_name="c", subcore_axis_name="s")

    @functools.partial(
        pl.kernel,
        mesh=mesh,
        out_type=[jax.ShapeDtypeStruct((N, C), jnp.float32) for _ in range(K)],
        scratch_types=[
            pltpu.VMEM_SHARED((ACC_ROWS, C), jnp.float32),
            pltpu.VMEM((NB_E, LB), jnp.int32),
            pltpu.VMEM((NB_E, LB), jnp.int32),
            pltpu.VMEM((LB, C), jnp.float32),
            pltpu.SemaphoreType.DMA,
        ],
    )
    def segsum(*refs):
        xs = refs[:K]
        src_h, dst_h, zero_h = refs[K:K + 3]
        outs = refs[K + 3:2 * K + 3]
        acc, src_v, dst_v, gbuf, sem = refs[2 * K + 3:]
        c = lax.axis_index("c")
        s = lax.axis_index("s")
        pltpu.sync_copy(src_h.at[s], src_v)
        pltpu.sync_copy(dst_h.at[s], dst_v)

        def do_chunk(x_hbm, out_hbm):
            pltpu.sync_copy(zero_h, acc.at[pl.ds(s * ZR, ZR)])
            plsc.subcore_barrier()

            def body(j, carry):
                pltpu.async_copy(x_hbm.at[src_v.at[j]], gbuf, sem).wait()
                pltpu.sync_copy(gbuf, acc.at[dst_v.at[j]], add=True)
                return carry

            lax.fori_loop(0, NB_E, body, 0)
            plsc.subcore_barrier()
            pltpu.sync_copy(acc.at[pl.ds(s * WR, WR)],
                            out_hbm.at[pl.ds(s * WR, WR)])
            plsc.subcore_barrier()

        for half in range(NC):
            @pl.when(c == half)
            def _half(half=half):
                for i in range(Khalf):
                    k = half * Khalf + i
                    do_chunk(xs[k], outs[k])

    return segsum


@functools.lru_cache(None)
def _p1_call(K, interpret=False):
    Win = K * C

    def body(*refs):
        i = pl.program_id(0)
        xs = refs[:K]
        ags = refs[K:2 * K]
        W1, b1 = refs[2 * K], refs[2 * K + 1]
        T_ref, S_ref = refs[2 * K + 2], refs[2 * K + 3]
        acc = jnp.zeros((BR, H), jnp.float32)
        for k in range(K):
            u = xs[k][...] + ags[k][...]
            acc = acc + jnp.dot(u, W1[k * C:(k + 1) * C, :],
                                preferred_element_type=jnp.float32)
        T = acc + b1[...]
        T_ref[...] = T
        Sb = jnp.concatenate(
            [jnp.sum(T, axis=0, keepdims=True),
             jnp.sum(T * T, axis=0, keepdims=True),
             jnp.zeros((6, H), jnp.float32)], axis=0)

        @pl.when(i == 0)
        def _():
            S_ref[...] = Sb

        @pl.when(i != 0)
        def _():
            S_ref[...] = S_ref[...] + Sb

    in_specs = (
        [pl.BlockSpec((BR, C), lambda i: (i, 0)) for _ in range(2 * K)]
        + [pl.BlockSpec((Win, H), lambda i: (0, 0)),
           pl.BlockSpec((1, H), lambda i: (0, 0))]
    )
    out_specs = [pl.BlockSpec((BR, H), lambda i: (i, 0)),
                 pl.BlockSpec((8, H), lambda i: (0, 0))]
    out_shape = [jax.ShapeDtypeStruct((N, H), jnp.float32),
                 jax.ShapeDtypeStruct((8, H), jnp.float32)]
    return pl.pallas_call(body, grid=(NBLK,), in_specs=in_specs,
                          out_specs=out_specs, out_shape=out_shape,
                          interpret=interpret)


def _bn_relu(T_blk, S_ref, g_ref, be_ref):
    mu = S_ref[0:1, :] * (1.0 / N)
    ex2 = S_ref[1:2, :] * (1.0 / N)
    inv = lax.rsqrt(ex2 - mu * mu + EPS)
    return jnp.maximum((T_blk - mu) * (inv * g_ref[...]) + be_ref[...], 0.0)


@functools.lru_cache(None)
def _p2_mid_call(interpret=False):
    def body(T_ref, S_ref, g_ref, be_ref, W2_ref, b2_ref, *h_refs):
        Tn = _bn_relu(T_ref[...], S_ref, g_ref, be_ref)
        R = jnp.dot(Tn, W2_ref[...],
                    preferred_element_type=jnp.float32) + b2_ref[...]
        for k in range(4):
            h_refs[k][...] = R[:, k * C:(k + 1) * C]

    in_specs = [pl.BlockSpec((BR, H), lambda i: (i, 0)),
                pl.BlockSpec((8, H), lambda i: (0, 0)),
                pl.BlockSpec((1, H), lambda i: (0, 0)),
                pl.BlockSpec((1, H), lambda i: (0, 0)),
                pl.BlockSpec((H, H), lambda i: (0, 0)),
                pl.BlockSpec((1, H), lambda i: (0, 0))]
    out_specs = [pl.BlockSpec((BR, C), lambda i: (i, 0)) for _ in range(4)]
    out_shape = [jax.ShapeDtypeStruct((N, C), jnp.float32) for _ in range(4)]
    return pl.pallas_call(body, grid=(NBLK,), in_specs=in_specs,
                          out_specs=out_specs, out_shape=out_shape,
                          interpret=interpret)


@functools.lru_cache(None)
def _p2_fin_call(interpret=False):
    def body(T_ref, S_ref, g_ref, be_ref, W2_ref, b2_ref, fcW_ref, fcb_ref,
             batch_ref, out_ref, Z_ref):
        i = pl.program_id(0)
        Tn = _bn_relu(T_ref[...], S_ref, g_ref, be_ref)
        w = jnp.dot(W2_ref[...], fcW_ref[...],
                    preferred_element_type=jnp.float32)
        y = jnp.dot(Tn, w, preferred_element_type=jnp.float32)
        y = y + jnp.dot(b2_ref[...], fcW_ref[...],
                        preferred_element_type=jnp.float32)
        col = lax.broadcasted_iota(jnp.int32, (BR, 128), 1)
        Q = jnp.where(col == 0, y, jnp.where(col == 1, 1.0, 0.0))
        b = batch_ref[0, 0, :]
        M = (b[:, None] == lax.broadcasted_iota(jnp.int32, (BR, G), 1)
             ).astype(jnp.float32)
        Z = lax.dot_general(M, Q, (((0,), (0,)), ((), ())),
                            preferred_element_type=jnp.float32)

        @pl.when(i == 0)
        def _():
            Z_ref[...] = Z
            out_ref[...] = jnp.zeros((G, 1), jnp.float32)

        @pl.when(i != 0)
        def _():
            Z_ref[...] = Z_ref[...] + Z

        @pl.when(i == NBLK - 1)
        def _():
            Zf = Z_ref[...]
            cnt = jnp.maximum(Zf[:, 1:2], 1.0)
            out_ref[...] = Zf[:, 0:1] / cnt + fcb_ref[...]

    in_specs = [pl.BlockSpec((BR, H), lambda i: (i, 0)),
                pl.BlockSpec((8, H), lambda i: (0, 0)),
                pl.BlockSpec((1, H), lambda i: (0, 0)),
                pl.BlockSpec((1, H), lambda i: (0, 0)),
                pl.BlockSpec((H, H), lambda i: (0, 0)),
                pl.BlockSpec((1, H), lambda i: (0, 0)),
                pl.BlockSpec((H, 128), lambda i: (0, 0)),
                pl.BlockSpec((1, 1), lambda i: (0, 0)),
                pl.BlockSpec((1, 1, BR), lambda i: (i, 0, 0))]
    out_specs = pl.BlockSpec((G, 1), lambda i: (0, 0))
    out_shape = jax.ShapeDtypeStruct((G, 1), jnp.float32)
    return pl.pallas_call(body, grid=(NBLK,), in_specs=in_specs,
                          out_specs=out_specs, out_shape=out_shape,
                          scratch_shapes=[pltpu.VMEM((G, 128), jnp.float32)],
                          interpret=interpret)


def kernel(x, edge_index, batch, W1a, b1a, g1a, be1a, W2a, b2a,
           W1b, b1b, g1b, be1b, W2b, b2b,
           W1c, b1c, g1c, be1c, W2c, b2c, fcW, fcb):
    f32 = jnp.float32
    x0 = x[:, :C]
    x1 = x[:, C:]
    E = edge_index.shape[1]
    pad = EPAD - E
    src = jnp.concatenate(
        [edge_index[0], jnp.zeros((pad,), jnp.int32)]).reshape(NT, NB_E, LB)
    dst = jnp.concatenate(
        [edge_index[1], jnp.full((pad,), N, jnp.int32)]).reshape(NT, NB_E, LB)
    zeros = jnp.zeros((ZR, C), f32)
    batch3 = batch.reshape(NBLK, 1, BR)
    r = lambda v: v.reshape(1, -1)

    seg2, seg4 = _segsum_call(2), _segsum_call(4)
    p1_2, p1_4 = _p1_call(2), _p1_call(4)
    p2m, p2f = _p2_mid_call(), _p2_fin_call()

    a0, a1 = seg2(x0, x1, src, dst, zeros)
    T, S = p1_2(x0, x1, a0, a1, W1a, r(b1a))
    h = p2m(T, S, r(g1a), r(be1a), W2a, r(b2a))
    a = seg4(*h, src, dst, zeros)
    T, S = p1_4(*h, *a, W1b, r(b1b))
    h = p2m(T, S, r(g1b), r(be1b), W2b, r(b2b))
    a = seg4(*h, src, dst, zeros)
    T, S = p1_4(*h, *a, W1c, r(b1c))
    fcWp = jnp.pad(fcW, ((0, 0), (0, 127)))
    return p2f(T, S, r(g1c), r(be1c), W2c, r(b2c), fcWp,
               fcb.reshape(1, 1), batch3)


# speedup vs baseline: 2.3174x; 2.3174x over previous
import functools

import jax
import jax.numpy as jnp
from jax import lax
from jax.experimental import pallas as pl
from jax.experimental.pallas import tpu as pltpu
from jax.experimental.pallas import tpu_sc as plsc

N = 10000
G = 64
H = 512
EPS = 1e-5
C = 128
LB = 128
NT = 16
NC = 2
NB_E = 80
EPT = NB_E * LB
EPAD = EPT * NT
ACC_ROWS = 10112
ZR = ACC_ROWS // NT
WR = 624
WTAIL = N - NT * WR
BR = 1000
NBLK = N // BR


@functools.lru_cache(None)
def _segsum_call(K):
    Khalf = K // 2
    mesh = plsc.VectorSubcoreMesh(core_axis_name="c", subcore_axis_name="s")

    @functools.partial(
        pl.kernel,
        mesh=mesh,
        out_type=[jax.ShapeDtypeStruct((N, C), jnp.float32) for _ in range(K)],
        scratch_types=[
            pltpu.VMEM_SHARED((ACC_ROWS, C), jnp.float32),
            pltpu.VMEM((NB_E, LB), jnp.int32),
            pltpu.VMEM((NB_E, LB), jnp.int32),
            pltpu.VMEM((LB, C), jnp.float32),
            pltpu.SemaphoreType.DMA,
        ],
    )
    def segsum(*refs):
        xs = refs[:K]
        src_h, dst_h, zero_h = refs[K:K + 3]
        outs = refs[K + 3:2 * K + 3]
        acc, src_v, dst_v, gbuf, sem = refs[2 * K + 3:]
        c = lax.axis_index("c")
        s = lax.axis_index("s")
        pltpu.sync_copy(src_h.at[s], src_v)
        pltpu.sync_copy(dst_h.at[s], dst_v)

        def do_chunk(x_hbm, out_hbm):
            pltpu.sync_copy(zero_h, acc.at[pl.ds(s * ZR, ZR)])
            plsc.subcore_barrier()

            def body(j, carry):
                pltpu.async_copy(x_hbm.at[src_v.at[j]], gbuf, sem).wait()
                pltpu.sync_copy(gbuf, acc.at[dst_v.at[j]], add=True)
                return carry

            lax.fori_loop(0, NB_E, body, 0)
            plsc.subcore_barrier()
            pltpu.sync_copy(acc.at[pl.ds(s * WR, WR)],
                            out_hbm.at[pl.ds(s * WR, WR)])

            @pl.when(s == NT - 1)
            def _tail():
                pltpu.sync_copy(acc.at[pl.ds(NT * WR, WTAIL)],
                                out_hbm.at[pl.ds(NT * WR, WTAIL)])

            plsc.subcore_barrier()

        for half in range(NC):
            @pl.when(c == half)
            def _half(half=half):
                for i in range(Khalf):
                    k = half * Khalf + i
                    do_chunk(xs[k], outs[k])

    return segsum


@functools.lru_cache(None)
def _p1_call(K, interpret=False):
    Win = K * C

    def body(*refs):
        i = pl.program_id(0)
        xs = refs[:K]
        ags = refs[K:2 * K]
        W1, b1 = refs[2 * K], refs[2 * K + 1]
        T_ref, S_ref = refs[2 * K + 2], refs[2 * K + 3]
        acc = jnp.zeros((BR, H), jnp.float32)
        for k in range(K):
            u = xs[k][...] + ags[k][...]
            acc = acc + jnp.dot(u, W1[k * C:(k + 1) * C, :],
                                preferred_element_type=jnp.float32)
        T = acc + b1[...]
        T_ref[...] = T
        Sb = jnp.concatenate(
            [jnp.sum(T, axis=0, keepdims=True),
             jnp.sum(T * T, axis=0, keepdims=True),
             jnp.zeros((6, H), jnp.float32)], axis=0)

        @pl.when(i == 0)
        def _():
            S_ref[...] = Sb

        @pl.when(i != 0)
        def _():
            S_ref[...] = S_ref[...] + Sb

    in_specs = (
        [pl.BlockSpec((BR, C), lambda i: (i, 0)) for _ in range(2 * K)]
        + [pl.BlockSpec((Win, H), lambda i: (0, 0)),
           pl.BlockSpec((1, H), lambda i: (0, 0))]
    )
    out_specs = [pl.BlockSpec((BR, H), lambda i: (i, 0)),
                 pl.BlockSpec((8, H), lambda i: (0, 0))]
    out_shape = [jax.ShapeDtypeStruct((N, H), jnp.float32),
                 jax.ShapeDtypeStruct((8, H), jnp.float32)]
    return pl.pallas_call(body, grid=(NBLK,), in_specs=in_specs,
                          out_specs=out_specs, out_shape=out_shape,
                          interpret=interpret)


def _bn_relu(T_blk, S_ref, g_ref, be_ref):
    mu = S_ref[0:1, :] * (1.0 / N)
    ex2 = S_ref[1:2, :] * (1.0 / N)
    inv = lax.rsqrt(ex2 - mu * mu + EPS)
    return jnp.maximum((T_blk - mu) * (inv * g_ref[...]) + be_ref[...], 0.0)


@functools.lru_cache(None)
def _p2_mid_call(interpret=False):
    def body(T_ref, S_ref, g_ref, be_ref, W2_ref, b2_ref, *h_refs):
        Tn = _bn_relu(T_ref[...], S_ref, g_ref, be_ref)
        R = jnp.dot(Tn, W2_ref[...],
                    preferred_element_type=jnp.float32) + b2_ref[...]
        for k in range(4):
            h_refs[k][...] = R[:, k * C:(k + 1) * C]

    in_specs = [pl.BlockSpec((BR, H), lambda i: (i, 0)),
                pl.BlockSpec((8, H), lambda i: (0, 0)),
                pl.BlockSpec((1, H), lambda i: (0, 0)),
                pl.BlockSpec((1, H), lambda i: (0, 0)),
                pl.BlockSpec((H, H), lambda i: (0, 0)),
                pl.BlockSpec((1, H), lambda i: (0, 0))]
    out_specs = [pl.BlockSpec((BR, C), lambda i: (i, 0)) for _ in range(4)]
    out_shape = [jax.ShapeDtypeStruct((N, C), jnp.float32) for _ in range(4)]
    return pl.pallas_call(body, grid=(NBLK,), in_specs=in_specs,
                          out_specs=out_specs, out_shape=out_shape,
                          interpret=interpret)


@functools.lru_cache(None)
def _p2_fin_call(interpret=False):
    def body(T_ref, S_ref, g_ref, be_ref, W2_ref, b2_ref, fcW_ref, fcb_ref,
             batch_ref, out_ref, Z_ref):
        i = pl.program_id(0)
        Tn = _bn_relu(T_ref[...], S_ref, g_ref, be_ref)
        w = jnp.dot(W2_ref[...], fcW_ref[...],
                    preferred_element_type=jnp.float32)
        y = jnp.dot(Tn, w, preferred_element_type=jnp.float32)
        y = y + jnp.dot(b2_ref[...], fcW_ref[...],
                        preferred_element_type=jnp.float32)
        col = lax.broadcasted_iota(jnp.int32, (BR, 128), 1)
        Q = jnp.where(col == 0, y, jnp.where(col == 1, 1.0, 0.0))
        b = batch_ref[0, 0, :]
        M = (b[:, None] == lax.broadcasted_iota(jnp.int32, (BR, G), 1)
             ).astype(jnp.float32)
        Z = lax.dot_general(M, Q, (((0,), (0,)), ((), ())),
                            preferred_element_type=jnp.float32)

        @pl.when(i == 0)
        def _():
            Z_ref[...] = Z
            out_ref[...] = jnp.zeros((G, 1), jnp.float32)

        @pl.when(i != 0)
        def _():
            Z_ref[...] = Z_ref[...] + Z

        @pl.when(i == NBLK - 1)
        def _():
            Zf = Z_ref[...]
            cnt = jnp.maximum(Zf[:, 1:2], 1.0)
            out_ref[...] = Zf[:, 0:1] / cnt + fcb_ref[...]

    in_specs = [pl.BlockSpec((BR, H), lambda i: (i, 0)),
                pl.BlockSpec((8, H), lambda i: (0, 0)),
                pl.BlockSpec((1, H), lambda i: (0, 0)),
                pl.BlockSpec((1, H), lambda i: (0, 0)),
                pl.BlockSpec((H, H), lambda i: (0, 0)),
                pl.BlockSpec((1, H), lambda i: (0, 0)),
                pl.BlockSpec((H, 128), lambda i: (0, 0)),
                pl.BlockSpec((1, 1), lambda i: (0, 0)),
                pl.BlockSpec((1, 1, BR), lambda i: (i, 0, 0))]
    out_specs = pl.BlockSpec((G, 1), lambda i: (0, 0))
    out_shape = jax.ShapeDtypeStruct((G, 1), jnp.float32)
    return pl.pallas_call(body, grid=(NBLK,), in_specs=in_specs,
                          out_specs=out_specs, out_shape=out_shape,
                          scratch_shapes=[pltpu.VMEM((G, 128), jnp.float32)],
                          interpret=interpret)


def kernel(x, edge_index, batch, W1a, b1a, g1a, be1a, W2a, b2a,
           W1b, b1b, g1b, be1b, W2b, b2b,
           W1c, b1c, g1c, be1c, W2c, b2c, fcW, fcb):
    f32 = jnp.float32
    x0 = x[:, :C]
    x1 = x[:, C:]
    E = edge_index.shape[1]
    pad = EPAD - E
    src = jnp.concatenate(
        [edge_index[0], jnp.zeros((pad,), jnp.int32)]).reshape(NT, NB_E, LB)
    dst = jnp.concatenate(
        [edge_index[1], jnp.full((pad,), N, jnp.int32)]).reshape(NT, NB_E, LB)
    zeros = jnp.zeros((ZR, C), f32)
    batch3 = batch.reshape(NBLK, 1, BR)
    r = lambda v: v.reshape(1, -1)

    seg2, seg4 = _segsum_call(2), _segsum_call(4)
    p1_2, p1_4 = _p1_call(2), _p1_call(4)
    p2m, p2f = _p2_mid_call(), _p2_fin_call()

    a0, a1 = seg2(x0, x1, src, dst, zeros)
    T, S = p1_2(x0, x1, a0, a1, W1a, r(b1a))
    h = p2m(T, S, r(g1a), r(be1a), W2a, r(b2a))
    a = seg4(*h, src, dst, zeros)
    T, S = p1_4(*h, *a, W1b, r(b1b))
    h = p2m(T, S, r(g1b), r(be1b), W2b, r(b2b))
    a = seg4(*h, src, dst, zeros)
    T, S = p1_4(*h, *a, W1c, r(b1c))
    fcWp = jnp.pad(fcW, ((0, 0), (0, 127)))
    return p2f(T, S, r(g1c), r(be1c), W2c, r(b2c), fcWp,
               fcb.reshape(1, 1), batch3)


# speedup vs baseline: 2.7704x; 1.1955x over previous
import functools

import jax
import jax.numpy as jnp
from jax import lax
from jax.experimental import pallas as pl
from jax.experimental.pallas import tpu as pltpu
from jax.experimental.pallas import tpu_sc as plsc

N = 10000
G = 64
H = 512
EPS = 1e-5
C = 128
LB = 128
NT = 16
NC = 2
NB_E = 80
EPT = NB_E * LB
EPAD = EPT * NT
ACC_ROWS = 10112
ZR = ACC_ROWS // NT
WR = 624
WTAIL = N - NT * WR
BR = 1000
NBLK = N // BR
NBUF = 2
IH = 40


@functools.lru_cache(None)
def _segsum_call(K):
    Khalf = K // 2
    mesh = plsc.VectorSubcoreMesh(core_axis_name="c", subcore_axis_name="s")

    @functools.partial(
        pl.kernel,
        mesh=mesh,
        out_type=[jax.ShapeDtypeStruct((N, C), jnp.float32) for _ in range(K)],
        scratch_types=[
            pltpu.VMEM_SHARED((ACC_ROWS, C), jnp.float32),
            pltpu.VMEM((IH, LB), jnp.int32),
            pltpu.VMEM((IH, LB), jnp.int32),
        ] + [pltpu.VMEM((LB, C), jnp.float32) for _ in range(NBUF)]
          + [pltpu.SemaphoreType.DMA for _ in range(NBUF)],
    )
    def segsum(*refs):
        xs = refs[:K]
        src_h, dst_h, zero_h = refs[K:K + 3]
        outs = refs[K + 3:2 * K + 3]
        acc, src_v, dst_v = refs[2 * K + 3:2 * K + 6]
        gb = refs[2 * K + 6:2 * K + 6 + NBUF]
        sems = refs[2 * K + 6 + NBUF:]
        c = lax.axis_index("c")
        s = lax.axis_index("s")

        def do_chunk(x_hbm, out_hbm):
            pltpu.sync_copy(zero_h, acc.at[pl.ds(s * ZR, ZR)])
            plsc.subcore_barrier()
            for stage in range(NB_E // IH):
                pltpu.sync_copy(src_h.at[s].at[pl.ds(stage * IH, IH)], src_v)
                pltpu.sync_copy(dst_h.at[s].at[pl.ds(stage * IH, IH)], dst_v)
                for b in range(NBUF):
                    pltpu.async_copy(x_hbm.at[src_v.at[b]], gb[b], sems[b])

                def body(j, carry):
                    for b in range(NBUF):
                        jj = j * NBUF + b
                        pltpu.make_async_copy(
                            x_hbm.at[src_v.at[jj]], gb[b], sems[b]).wait()
                        pltpu.sync_copy(gb[b], acc.at[dst_v.at[jj]], add=True)

                        @pl.when(jj + NBUF < IH)
                        def _():
                            pltpu.async_copy(
                                x_hbm.at[src_v.at[jj + NBUF]], gb[b], sems[b])
                    return carry

                lax.fori_loop(0, IH // NBUF, body, 0)
            plsc.subcore_barrier()
            pltpu.sync_copy(acc.at[pl.ds(s * WR, WR)],
                            out_hbm.at[pl.ds(s * WR, WR)])

            @pl.when(s == NT - 1)
            def _tail():
                pltpu.sync_copy(acc.at[pl.ds(NT * WR, WTAIL)],
                                out_hbm.at[pl.ds(NT * WR, WTAIL)])

            plsc.subcore_barrier()

        for half in range(NC):
            @pl.when(c == half)
            def _half(half=half):
                for i in range(Khalf):
                    k = half * Khalf + i
                    do_chunk(xs[k], outs[k])

    return segsum


@functools.lru_cache(None)
def _p1_call(K, interpret=False):
    Win = K * C

    def body(*refs):
        i = pl.program_id(0)
        xs = refs[:K]
        ags = refs[K:2 * K]
        W1, b1 = refs[2 * K], refs[2 * K + 1]
        T_ref, S_ref = refs[2 * K + 2], refs[2 * K + 3]
        acc = jnp.zeros((BR, H), jnp.float32)
        for k in range(K):
            u = xs[k][...] + ags[k][...]
            acc = acc + jnp.dot(u, W1[k * C:(k + 1) * C, :],
                                preferred_element_type=jnp.float32)
        T = acc + b1[...]
        T_ref[...] = T
        Sb = jnp.concatenate(
            [jnp.sum(T, axis=0, keepdims=True),
             jnp.sum(T * T, axis=0, keepdims=True),
             jnp.zeros((6, H), jnp.float32)], axis=0)

        @pl.when(i == 0)
        def _():
            S_ref[...] = Sb

        @pl.when(i != 0)
        def _():
            S_ref[...] = S_ref[...] + Sb

    in_specs = (
        [pl.BlockSpec((BR, C), lambda i: (i, 0)) for _ in range(2 * K)]
        + [pl.BlockSpec((Win, H), lambda i: (0, 0)),
           pl.BlockSpec((1, H), lambda i: (0, 0))]
    )
    out_specs = [pl.BlockSpec((BR, H), lambda i: (i, 0)),
                 pl.BlockSpec((8, H), lambda i: (0, 0))]
    out_shape = [jax.ShapeDtypeStruct((N, H), jnp.float32),
                 jax.ShapeDtypeStruct((8, H), jnp.float32)]
    return pl.pallas_call(body, grid=(NBLK,), in_specs=in_specs,
                          out_specs=out_specs, out_shape=out_shape,
                          interpret=interpret)


def _bn_relu(T_blk, S_ref, g_ref, be_ref):
    mu = S_ref[0:1, :] * (1.0 / N)
    ex2 = S_ref[1:2, :] * (1.0 / N)
    inv = lax.rsqrt(ex2 - mu * mu + EPS)
    return jnp.maximum((T_blk - mu) * (inv * g_ref[...]) + be_ref[...], 0.0)


@functools.lru_cache(None)
def _p2_mid_call(interpret=False):
    def body(T_ref, S_ref, g_ref, be_ref, W2_ref, b2_ref, *h_refs):
        Tn = _bn_relu(T_ref[...], S_ref, g_ref, be_ref)
        R = jnp.dot(Tn, W2_ref[...],
                    preferred_element_type=jnp.float32) + b2_ref[...]
        for k in range(4):
            h_refs[k][...] = R[:, k * C:(k + 1) * C]

    in_specs = [pl.BlockSpec((BR, H), lambda i: (i, 0)),
                pl.BlockSpec((8, H), lambda i: (0, 0)),
                pl.BlockSpec((1, H), lambda i: (0, 0)),
                pl.BlockSpec((1, H), lambda i: (0, 0)),
                pl.BlockSpec((H, H), lambda i: (0, 0)),
                pl.BlockSpec((1, H), lambda i: (0, 0))]
    out_specs = [pl.BlockSpec((BR, C), lambda i: (i, 0)) for _ in range(4)]
    out_shape = [jax.ShapeDtypeStruct((N, C), jnp.float32) for _ in range(4)]
    return pl.pallas_call(body, grid=(NBLK,), in_specs=in_specs,
                          out_specs=out_specs, out_shape=out_shape,
                          interpret=interpret)


@functools.lru_cache(None)
def _p2_fin_call(interpret=False):
    def body(T_ref, S_ref, g_ref, be_ref, W2_ref, b2_ref, fcW_ref, fcb_ref,
             batch_ref, out_ref, Z_ref):
        i = pl.program_id(0)
        Tn = _bn_relu(T_ref[...], S_ref, g_ref, be_ref)
        w = jnp.dot(W2_ref[...], fcW_ref[...],
                    preferred_element_type=jnp.float32)
        y = jnp.dot(Tn, w, preferred_element_type=jnp.float32)
        y = y + jnp.dot(b2_ref[...], fcW_ref[...],
                        preferred_element_type=jnp.float32)
        col = lax.broadcasted_iota(jnp.int32, (BR, 128), 1)
        Q = jnp.where(col == 0, y, jnp.where(col == 1, 1.0, 0.0))
        b = batch_ref[0, 0, :]
        M = (b[:, None] == lax.broadcasted_iota(jnp.int32, (BR, G), 1)
             ).astype(jnp.float32)
        Z = lax.dot_general(M, Q, (((0,), (0,)), ((), ())),
                            preferred_element_type=jnp.float32)

        @pl.when(i == 0)
        def _():
            Z_ref[...] = Z
            out_ref[...] = jnp.zeros((G, 1), jnp.float32)

        @pl.when(i != 0)
        def _():
            Z_ref[...] = Z_ref[...] + Z

        @pl.when(i == NBLK - 1)
        def _():
            Zf = Z_ref[...]
            cnt = jnp.maximum(Zf[:, 1:2], 1.0)
            out_ref[...] = Zf[:, 0:1] / cnt + fcb_ref[...]

    in_specs = [pl.BlockSpec((BR, H), lambda i: (i, 0)),
                pl.BlockSpec((8, H), lambda i: (0, 0)),
                pl.BlockSpec((1, H), lambda i: (0, 0)),
                pl.BlockSpec((1, H), lambda i: (0, 0)),
                pl.BlockSpec((H, H), lambda i: (0, 0)),
                pl.BlockSpec((1, H), lambda i: (0, 0)),
                pl.BlockSpec((H, 128), lambda i: (0, 0)),
                pl.BlockSpec((1, 1), lambda i: (0, 0)),
                pl.BlockSpec((1, 1, BR), lambda i: (i, 0, 0))]
    out_specs = pl.BlockSpec((G, 1), lambda i: (0, 0))
    out_shape = jax.ShapeDtypeStruct((G, 1), jnp.float32)
    return pl.pallas_call(body, grid=(NBLK,), in_specs=in_specs,
                          out_specs=out_specs, out_shape=out_shape,
                          scratch_shapes=[pltpu.VMEM((G, 128), jnp.float32)],
                          interpret=interpret)


def kernel(x, edge_index, batch, W1a, b1a, g1a, be1a, W2a, b2a,
           W1b, b1b, g1b, be1b, W2b, b2b,
           W1c, b1c, g1c, be1c, W2c, b2c, fcW, fcb):
    f32 = jnp.float32
    x0 = x[:, :C]
    x1 = x[:, C:]
    E = edge_index.shape[1]
    pad = EPAD - E
    src = jnp.concatenate(
        [edge_index[0], jnp.zeros((pad,), jnp.int32)]).reshape(NT, NB_E, LB)
    dst = jnp.concatenate(
        [edge_index[1], jnp.full((pad,), N, jnp.int32)]).reshape(NT, NB_E, LB)
    zeros = jnp.zeros((ZR, C), f32)
    batch3 = batch.reshape(NBLK, 1, BR)
    r = lambda v: v.reshape(1, -1)

    seg2, seg4 = _segsum_call(2), _segsum_call(4)
    p1_2, p1_4 = _p1_call(2), _p1_call(4)
    p2m, p2f = _p2_mid_call(), _p2_fin_call()

    a0, a1 = seg2(x0, x1, src, dst, zeros)
    T, S = p1_2(x0, x1, a0, a1, W1a, r(b1a))
    h = p2m(T, S, r(g1a), r(be1a), W2a, r(b2a))
    a = seg4(*h, src, dst, zeros)
    T, S = p1_4(*h, *a, W1b, r(b1b))
    h = p2m(T, S, r(g1b), r(be1b), W2b, r(b2b))
    a = seg4(*h, src, dst, zeros)
    T, S = p1_4(*h, *a, W1c, r(b1c))
    fcWp = jnp.pad(fcW, ((0, 0), (0, 127)))
    return p2f(T, S, r(g1c), r(be1c), W2c, r(b2c), fcWp,
               fcb.reshape(1, 1), batch3)
